# baseline (speedup 1.0000x reference)
import jax
import jax.numpy as jnp
from jax.experimental import pallas as pl
from jax.experimental.pallas import tpu as pltpu

_INV_SCALE = 1.0 / 128.0
_HALF_INV_SCALE = 1.0 / 256.0
_SCORE_CLIP = 100.0
_MIN_SCORE_THRESH = 0.75


def _body(anchors_ref, raw_ref, score_ref, det_ref, mask_ref):
    ax = anchors_ref[0:1, :]
    ay = anchors_ref[1:2, :]
    sx = anchors_ref[2:3, :] * _INV_SCALE
    sy = anchors_ref[3:4, :] * _INV_SCALE
    hx = anchors_ref[2:3, :] * _HALF_INV_SCALE
    hy = anchors_ref[3:4, :] * _HALF_INV_SCALE

    xc = raw_ref[:, 0, :] * sx + ax
    yc = raw_ref[:, 1, :] * sy + ay
    w2 = raw_ref[:, 2, :] * hx
    h2 = raw_ref[:, 3, :] * hy
    det_ref[0, :, :] = yc - h2
    det_ref[1, :, :] = xc - w2
    det_ref[2, :, :] = yc + h2
    det_ref[3, :, :] = xc + w2
    for k in range(6):
        det_ref[4 + 2 * k, :, :] = raw_ref[:, 4 + 2 * k, :] * sx + ax
        det_ref[5 + 2 * k, :, :] = raw_ref[:, 5 + 2 * k, :] * sy + ay

    sig = 1.0 / (1.0 + jnp.exp(-jnp.clip(score_ref[:, 0, :], -_SCORE_CLIP,
                                         _SCORE_CLIP)))
    det_ref[16, :, :] = sig
    mask_ref[...] = sig >= _MIN_SCORE_THRESH


@jax.jit
def kernel(raw_box_tensor, raw_score_tensor, anchors):
    B, A, C = raw_box_tensor.shape
    blk = 128
    raw_t = jnp.transpose(raw_box_tensor, (0, 2, 1))
    anchors_t = jnp.transpose(anchors, (1, 0))
    score_t = jnp.transpose(raw_score_tensor, (0, 2, 1))
    det_t, mask = pl.pallas_call(
        _body,
        grid=(B // blk,),
        in_specs=[
            pl.BlockSpec((4, A), lambda i: (0, 0)),
            pl.BlockSpec((blk, C, A), lambda i: (i, 0, 0)),
            pl.BlockSpec((blk, 1, A), lambda i: (i, 0, 0)),
        ],
        out_specs=[
            pl.BlockSpec((C + 1, blk, A), lambda i: (0, i, 0)),
            pl.BlockSpec((blk, A), lambda i: (i, 0)),
        ],
        out_shape=[
            jax.ShapeDtypeStruct((C + 1, B, A), jnp.float32),
            jax.ShapeDtypeStruct((B, A), jnp.bool_),
        ],
        compiler_params=pltpu.CompilerParams(
            dimension_semantics=("arbitrary",),
        ),
    )(anchors_t, raw_t, score_t)
    det = jnp.transpose(det_t, (1, 2, 0))
    return det, mask


# speedup vs baseline: 1.0271x; 1.0271x over previous
import jax
import jax.numpy as jnp
from jax.experimental import pallas as pl
from jax.experimental.pallas import tpu as pltpu

_INV_SCALE = 1.0 / 128.0
_HALF_INV_SCALE = 1.0 / 256.0
_SCORE_CLIP = 100.0
_MIN_SCORE_THRESH = 0.75


def _body(anchors_ref, raw_ref, score_ref, det_ref, mask_ref):
    ax = anchors_ref[0:1, :]
    ay = anchors_ref[1:2, :]
    sx = anchors_ref[2:3, :] * _INV_SCALE
    sy = anchors_ref[3:4, :] * _INV_SCALE
    hx = anchors_ref[2:3, :] * _HALF_INV_SCALE
    hy = anchors_ref[3:4, :] * _HALF_INV_SCALE

    xc = raw_ref[:, 0, :] * sx + ax
    yc = raw_ref[:, 1, :] * sy + ay
    w2 = raw_ref[:, 2, :] * hx
    h2 = raw_ref[:, 3, :] * hy
    det_ref[0, :, :] = yc - h2
    det_ref[1, :, :] = xc - w2
    det_ref[2, :, :] = yc + h2
    det_ref[3, :, :] = xc + w2
    for k in range(6):
        det_ref[4 + 2 * k, :, :] = raw_ref[:, 4 + 2 * k, :] * sx + ax
        det_ref[5 + 2 * k, :, :] = raw_ref[:, 5 + 2 * k, :] * sy + ay

    sig = 1.0 / (1.0 + jnp.exp(-jnp.clip(score_ref[:, 0, :], -_SCORE_CLIP,
                                         _SCORE_CLIP)))
    det_ref[16, :, :] = sig
    mask_ref[...] = (sig >= _MIN_SCORE_THRESH).astype(jnp.int8)


@jax.jit
def kernel(raw_box_tensor, raw_score_tensor, anchors):
    B, A, C = raw_box_tensor.shape
    blk = 128
    raw_t = jnp.transpose(raw_box_tensor, (0, 2, 1))
    anchors_t = jnp.transpose(anchors, (1, 0))
    score_t = jnp.transpose(raw_score_tensor, (0, 2, 1))
    det_t, mask8 = pl.pallas_call(
        _body,
        grid=(B // blk,),
        in_specs=[
            pl.BlockSpec((4, A), lambda i: (0, 0)),
            pl.BlockSpec((blk, C, A), lambda i: (i, 0, 0)),
            pl.BlockSpec((blk, 1, A), lambda i: (i, 0, 0)),
        ],
        out_specs=[
            pl.BlockSpec((C + 1, blk, A), lambda i: (0, i, 0)),
            pl.BlockSpec((blk, A), lambda i: (i, 0)),
        ],
        out_shape=[
            jax.ShapeDtypeStruct((C + 1, B, A), jnp.float32),
            jax.ShapeDtypeStruct((B, A), jnp.int8),
        ],
        compiler_params=pltpu.CompilerParams(
            dimension_semantics=("arbitrary",),
        ),
    )(anchors_t, raw_t, score_t)
    det = jnp.transpose(det_t, (1, 2, 0))
    return det, mask8.astype(jnp.bool_)


# speedup vs baseline: 1.0493x; 1.0216x over previous
import jax
import jax.numpy as jnp
from jax.experimental import pallas as pl
from jax.experimental.pallas import tpu as pltpu

_INV_SCALE = 1.0 / 128.0
_HALF_INV_SCALE = 1.0 / 256.0
_SCORE_CLIP = 100.0
_MIN_SCORE_THRESH = 0.75


def _body(anchors_ref, raw_ref, score_ref, det_ref, mask_ref):
    ax = anchors_ref[0:1, :]
    ay = anchors_ref[1:2, :]
    sx = anchors_ref[2:3, :] * _INV_SCALE
    sy = anchors_ref[3:4, :] * _INV_SCALE
    hx = anchors_ref[2:3, :] * _HALF_INV_SCALE
    hy = anchors_ref[3:4, :] * _HALF_INV_SCALE

    xc = raw_ref[:, 0, :] * sx + ax
    yc = raw_ref[:, 1, :] * sy + ay
    w2 = raw_ref[:, 2, :] * hx
    h2 = raw_ref[:, 3, :] * hy
    det_ref[0, :, :] = yc - h2
    det_ref[1, :, :] = xc - w2
    det_ref[2, :, :] = yc + h2
    det_ref[3, :, :] = xc + w2
    for k in range(6):
        det_ref[4 + 2 * k, :, :] = raw_ref[:, 4 + 2 * k, :] * sx + ax
        det_ref[5 + 2 * k, :, :] = raw_ref[:, 5 + 2 * k, :] * sy + ay

    sig = 1.0 / (1.0 + jnp.exp(-jnp.clip(score_ref[:, 0, :], -_SCORE_CLIP,
                                         _SCORE_CLIP)))
    det_ref[16, :, :] = sig
    mask_ref[...] = (sig >= _MIN_SCORE_THRESH).astype(jnp.int8)


@jax.jit
def kernel(raw_box_tensor, raw_score_tensor, anchors):
    B, A, C = raw_box_tensor.shape
    blk = 160
    raw_t = jnp.transpose(raw_box_tensor, (0, 2, 1))
    anchors_t = jnp.transpose(anchors, (1, 0))
    score_t = jnp.transpose(raw_score_tensor, (0, 2, 1))
    det_t, mask8 = pl.pallas_call(
        _body,
        grid=(pl.cdiv(B, blk),),
        in_specs=[
            pl.BlockSpec((4, A), lambda i: (0, 0)),
            pl.BlockSpec((blk, C, A), lambda i: (i, 0, 0)),
            pl.BlockSpec((blk, 1, A), lambda i: (i, 0, 0)),
        ],
        out_specs=[
            pl.BlockSpec((C + 1, blk, A), lambda i: (0, i, 0)),
            pl.BlockSpec((blk, A), lambda i: (i, 0)),
        ],
        out_shape=[
            jax.ShapeDtypeStruct((C + 1, B, A), jnp.float32),
            jax.ShapeDtypeStruct((B, A), jnp.int8),
        ],
        compiler_params=pltpu.CompilerParams(
            dimension_semantics=("arbitrary",),
        ),
    )(anchors_t, raw_t, score_t)
    det = jnp.transpose(det_t, (1, 2, 0))
    return det, mask8.astype(jnp.bool_)
